# initial kernel scaffold (unmeasured)
import functools

import jax
import jax.numpy as jnp
from jax import lax
from jax.experimental import pallas as pl
from jax.experimental.pallas import tpu as pltpu

W = 8
B_LOC = 2
SQ = 512
SKV = 512
H_LOC = 8
DH = 64
D = 768


def kernel(x, Wq, K_ext, V_ext, Wo):
    def body(x_ref, wq_ref, k_hbm, v_hbm, wo_ref, out_ref,
             wq_all, wo_all, kv_k, kv_v, bias_ref,
             copy_sem, kv_sems, wq_ssem, wq_rsem, wo_ssem, wo_rsem):
        my = lax.axis_index("i")
        right = lax.rem(my + 1, W)
        left = lax.rem(my + W - 1, W)

        own = []
        for h in range(H_LOC):
            c = pltpu.make_async_copy(
                wq_ref.at[:, h * DH:(h + 1) * DH], wq_all.at[my, h], copy_sem)
            c.start()
            own.append(c)
            c = pltpu.make_async_copy(
                wo_ref.at[h * DH:(h + 1) * DH, :], wo_all.at[my, h], copy_sem)
            c.start()
            own.append(c)

        def issue_kv(jj, slot):
            waits = []
            for b in range(B_LOC):
                bg = my * B_LOC + b
                for h in range(H_LOC):
                    hg = jj * H_LOC + h
                    for hbm, buf in ((k_hbm, kv_k), (v_hbm, kv_v)):
                        c = pltpu.make_async_copy(
                            hbm.at[bg, :, hg, :], buf.at[slot, b, h],
                            kv_sems.at[slot])
                        c.start()
                        waits.append(c)
            return waits

        kv_waits = issue_kv(my, 0)

        qi = lax.broadcasted_iota(jnp.int32, (SQ, SKV), 0)
        ki = lax.broadcasted_iota(jnp.int32, (SQ, SKV), 1)
        mask = (jnp.abs(qi - ki) <= 128) | (ki < 32) | (qi < 32)
        bias_ref[...] = jnp.where(mask, 0.0, -1e9).astype(jnp.float32)

        out_ref[...] = jnp.zeros((B_LOC, SQ, D), jnp.float32)

        bar = pltpu.get_barrier_semaphore()
        for nbr in (left, right):
            pl.semaphore_signal(bar, inc=1, device_id=(nbr,),
                                device_id_type=pl.DeviceIdType.MESH)
        pl.semaphore_wait(bar, 2)

        for c in own:
            c.wait()

        prev_rdma = []
        for hop in range(W):
            jj = lax.rem(my - hop + W, W)
            slot = hop % 2

            for r in prev_rdma:
                r.wait()

            if hop < W - 1:
                r1 = pltpu.make_async_remote_copy(
                    src_ref=wq_all.at[jj], dst_ref=wq_all.at[jj],
                    send_sem=wq_ssem.at[hop], recv_sem=wq_rsem.at[hop],
                    device_id=(right,), device_id_type=pl.DeviceIdType.MESH)
                r2 = pltpu.make_async_remote_copy(
                    src_ref=wo_all.at[jj], dst_ref=wo_all.at[jj],
                    send_sem=wo_ssem.at[hop], recv_sem=wo_rsem.at[hop],
                    device_id=(right,), device_id_type=pl.DeviceIdType.MESH)
                r1.start()
                r2.start()
                prev_rdma = [r1, r2]

            for c in kv_waits:
                c.wait()
            if hop < W - 1:
                kv_waits = issue_kv(lax.rem(my - hop - 1 + W, W), 1 - slot)

            def pair(idx, _, jj=jj, slot=slot):
                b = idx // H_LOC
                h = idx % H_LOC
                q = jnp.dot(x_ref[b], wq_all[jj, h],
                            preferred_element_type=jnp.float32)
                k = kv_k[slot, b, h]
                v = kv_v[slot, b, h]
                s = lax.dot_general(q, k, (((1,), (1,)), ((), ())),
                                    preferred_element_type=jnp.float32)
                s = s * 0.125 + bias_ref[...]
                m = jnp.max(s, axis=1, keepdims=True)
                w = jnp.exp(s - m)
                w = w / jnp.sum(w, axis=1, keepdims=True)
                ctx = jnp.dot(w, v, preferred_element_type=jnp.float32)
                out_ref[b] = out_ref[b] + jnp.dot(
                    ctx, wo_all[jj, h], preferred_element_type=jnp.float32)
                return 0

            lax.fori_loop(0, B_LOC * H_LOC, pair, 0)

        @functools.partial(pl.run_scoped,
                           exit_bar=pltpu.SemaphoreType.REGULAR)
        def _(exit_bar):
            for nbr in (left, right):
                pl.semaphore_signal(exit_bar, inc=1, device_id=(nbr,),
                                    device_id_type=pl.DeviceIdType.MESH)
            pl.semaphore_wait(exit_bar, 2)

    return pl.pallas_call(
        body,
        out_shape=jax.ShapeDtypeStruct((B_LOC, SQ, D), jnp.float32),
        in_specs=[
            pl.BlockSpec(memory_space=pltpu.VMEM),
            pl.BlockSpec(memory_space=pltpu.VMEM),
            pl.BlockSpec(memory_space=pltpu.ANY),
            pl.BlockSpec(memory_space=pltpu.ANY),
            pl.BlockSpec(memory_space=pltpu.VMEM),
        ],
        out_specs=pl.BlockSpec(memory_space=pltpu.VMEM),
        scratch_shapes=[
            pltpu.VMEM((W, H_LOC, D, DH), jnp.float32),
            pltpu.VMEM((W, H_LOC, DH, D), jnp.float32),
            pltpu.VMEM((2, B_LOC, H_LOC, SKV, DH), jnp.float32),
            pltpu.VMEM((2, B_LOC, H_LOC, SKV, DH), jnp.float32),
            pltpu.VMEM((SQ, SKV), jnp.float32),
            pltpu.SemaphoreType.DMA,
            pltpu.SemaphoreType.DMA((2,)),
            pltpu.SemaphoreType.DMA((W - 1,)),
            pltpu.SemaphoreType.DMA((W - 1,)),
            pltpu.SemaphoreType.DMA((W - 1,)),
            pltpu.SemaphoreType.DMA((W - 1,)),
        ],
        compiler_params=pltpu.CompilerParams(collective_id=0),
    )(x, Wq, K_ext, V_ext, Wo)


# baseline (device time: 871168 ns/iter reference)
import functools

import jax
import jax.numpy as jnp
from jax import lax
from jax.experimental import pallas as pl
from jax.experimental.pallas import tpu as pltpu

W = 8
B_LOC = 2
SQ = 512
SKV = 512
H_LOC = 8
DH = 64
D = 768


def kernel(x, Wq, K_ext, V_ext, Wo):
    def body(x_ref, wq_ref, k_hbm, v_hbm, wo_ref, out_ref,
             wq_all, wo_all, kv_k, kv_v, bias_ref,
             kv_sems, wq_ssem, wq_rsem, wo_ssem, wo_rsem):
        my = lax.axis_index("i")
        right = lax.rem(my + 1, W)
        left = lax.rem(my + W - 1, W)

        wq_v = wq_ref[...]
        wo_v = wo_ref[...]
        for h in range(H_LOC):
            wq_all[my, h] = wq_v[:, h * DH:(h + 1) * DH]
            wo_all[my, h] = wo_v[h * DH:(h + 1) * DH, :]

        def issue_kv(jj, slot):
            waits = []
            for b in range(B_LOC):
                bg = my * B_LOC + b
                for h in range(H_LOC):
                    hg = jj * H_LOC + h
                    for hbm, buf in ((k_hbm, kv_k), (v_hbm, kv_v)):
                        c = pltpu.make_async_copy(
                            hbm.at[bg, :, hg, :], buf.at[slot, b, h],
                            kv_sems.at[slot])
                        c.start()
                        waits.append(c)
            return waits

        kv_waits = issue_kv(my, 0)

        qi = lax.broadcasted_iota(jnp.int32, (SQ, SKV), 0)
        ki = lax.broadcasted_iota(jnp.int32, (SQ, SKV), 1)
        mask = (jnp.abs(qi - ki) <= 128) | (ki < 32) | (qi < 32)
        bias_ref[...] = jnp.where(mask, 0.0, -1e9).astype(jnp.float32)

        out_ref[...] = jnp.zeros((B_LOC, SQ, D), jnp.float32)

        bar = pltpu.get_barrier_semaphore()
        for nbr in (left, right):
            pl.semaphore_signal(bar, inc=1, device_id=(nbr,),
                                device_id_type=pl.DeviceIdType.MESH)
        pl.semaphore_wait(bar, 2)

        prev_rdma = []
        for hop in range(W):
            jj = lax.rem(my - hop + W, W)
            slot = hop % 2

            for r in prev_rdma:
                r.wait()

            if hop < W - 1:
                r1 = pltpu.make_async_remote_copy(
                    src_ref=wq_all.at[jj], dst_ref=wq_all.at[jj],
                    send_sem=wq_ssem.at[hop], recv_sem=wq_rsem.at[hop],
                    device_id=(right,), device_id_type=pl.DeviceIdType.MESH)
                r2 = pltpu.make_async_remote_copy(
                    src_ref=wo_all.at[jj], dst_ref=wo_all.at[jj],
                    send_sem=wo_ssem.at[hop], recv_sem=wo_rsem.at[hop],
                    device_id=(right,), device_id_type=pl.DeviceIdType.MESH)
                r1.start()
                r2.start()
                prev_rdma = [r1, r2]

            for c in kv_waits:
                c.wait()
            if hop < W - 1:
                kv_waits = issue_kv(lax.rem(my - hop - 1 + W, W), 1 - slot)

            def pair(idx, _, jj=jj, slot=slot):
                b = idx // H_LOC
                h = idx % H_LOC
                q = jnp.dot(x_ref[b], wq_all[jj, h],
                            preferred_element_type=jnp.float32)
                k = kv_k[slot, b, h]
                v = kv_v[slot, b, h]
                s = lax.dot_general(q, k, (((1,), (1,)), ((), ())),
                                    preferred_element_type=jnp.float32)
                s = s * 0.125 + bias_ref[...]
                m = jnp.max(s, axis=1, keepdims=True)
                w = jnp.exp(s - m)
                w = w / jnp.sum(w, axis=1, keepdims=True)
                ctx = jnp.dot(w, v, preferred_element_type=jnp.float32)
                out_ref[b] = out_ref[b] + jnp.dot(
                    ctx, wo_all[jj, h], preferred_element_type=jnp.float32)
                return 0

            lax.fori_loop(0, B_LOC * H_LOC, pair, 0)

        @functools.partial(pl.run_scoped,
                           exit_bar=pltpu.SemaphoreType.REGULAR)
        def _(exit_bar):
            for nbr in (left, right):
                pl.semaphore_signal(exit_bar, inc=1, device_id=(nbr,),
                                    device_id_type=pl.DeviceIdType.MESH)
            pl.semaphore_wait(exit_bar, 2)

    return pl.pallas_call(
        body,
        out_shape=jax.ShapeDtypeStruct((B_LOC, SQ, D), jnp.float32),
        in_specs=[
            pl.BlockSpec(memory_space=pltpu.VMEM),
            pl.BlockSpec(memory_space=pltpu.VMEM),
            pl.BlockSpec(memory_space=pltpu.MemorySpace.HBM),
            pl.BlockSpec(memory_space=pltpu.MemorySpace.HBM),
            pl.BlockSpec(memory_space=pltpu.VMEM),
        ],
        out_specs=pl.BlockSpec(memory_space=pltpu.VMEM),
        scratch_shapes=[
            pltpu.VMEM((W, H_LOC, D, DH), jnp.float32),
            pltpu.VMEM((W, H_LOC, DH, D), jnp.float32),
            pltpu.VMEM((2, B_LOC, H_LOC, SKV, DH), jnp.float32),
            pltpu.VMEM((2, B_LOC, H_LOC, SKV, DH), jnp.float32),
            pltpu.VMEM((SQ, SKV), jnp.float32),
            pltpu.SemaphoreType.DMA((2,)),
            pltpu.SemaphoreType.DMA((W - 1,)),
            pltpu.SemaphoreType.DMA((W - 1,)),
            pltpu.SemaphoreType.DMA((W - 1,)),
            pltpu.SemaphoreType.DMA((W - 1,)),
        ],
        compiler_params=pltpu.CompilerParams(
            collective_id=0, vmem_limit_bytes=100 * 1024 * 1024),
    )(x, Wq, K_ext, V_ext, Wo)


# device time: 739428 ns/iter; 1.1782x vs baseline; 1.1782x over previous
import functools

import jax
import jax.numpy as jnp
from jax import lax
from jax.experimental import pallas as pl
from jax.experimental.pallas import tpu as pltpu

W = 8
B_LOC = 2
SQ = 512
SKV = 512
H_LOC = 8
DH = 64
D = 768


def kernel(x, Wq, K_ext, V_ext, Wo):
    def body(x_ref, wq_ref, k_hbm, v_hbm, wo_ref, out_ref,
             wq_all, wo_all, kv_k, kv_v, bias_ref, ctx_buf,
             kv_sems, wq_ssem, wq_rsem, wo_ssem, wo_rsem):
        my = lax.axis_index("i")
        right = lax.rem(my + 1, W)
        left = lax.rem(my + W - 1, W)

        wq_all[my] = wq_ref[...] * 0.125
        wo_all[my] = wo_ref[...]

        def issue_kv(jj, slot):
            waits = []
            for b in range(B_LOC):
                bg = my * B_LOC + b
                for h in range(H_LOC):
                    hg = jj * H_LOC + h
                    for hbm, buf in ((k_hbm, kv_k), (v_hbm, kv_v)):
                        c = pltpu.make_async_copy(
                            hbm.at[bg, :, hg, :], buf.at[slot, b, h],
                            kv_sems.at[slot])
                        c.start()
                        waits.append(c)
            return waits

        kv_waits = issue_kv(my, 0)

        qi = lax.broadcasted_iota(jnp.int32, (SQ, SKV), 0)
        ki = lax.broadcasted_iota(jnp.int32, (SQ, SKV), 1)
        mask = (jnp.abs(qi - ki) <= 128) | (ki < 32) | (qi < 32)
        bias_ref[...] = jnp.where(mask, 0.0, -1e9).astype(jnp.float32)

        bar = pltpu.get_barrier_semaphore()
        for nbr in (left, right):
            pl.semaphore_signal(bar, inc=1, device_id=(nbr,),
                                device_id_type=pl.DeviceIdType.MESH)
        pl.semaphore_wait(bar, 2)

        prev_rdma = []
        for hop in range(W):
            jj = lax.rem(my - hop + W, W)
            slot = hop % 2

            for r in prev_rdma:
                r.wait()

            if hop < W - 1:
                r1 = pltpu.make_async_remote_copy(
                    src_ref=wq_all.at[jj], dst_ref=wq_all.at[jj],
                    send_sem=wq_ssem.at[hop], recv_sem=wq_rsem.at[hop],
                    device_id=(right,), device_id_type=pl.DeviceIdType.MESH)
                r2 = pltpu.make_async_remote_copy(
                    src_ref=wo_all.at[jj], dst_ref=wo_all.at[jj],
                    send_sem=wo_ssem.at[hop], recv_sem=wo_rsem.at[hop],
                    device_id=(right,), device_id_type=pl.DeviceIdType.MESH)
                r1.start()
                r2.start()
                prev_rdma = [r1, r2]

            for c in kv_waits:
                c.wait()
            if hop < W - 1:
                kv_waits = issue_kv(lax.rem(my - hop - 1 + W, W), 1 - slot)

            wq_j = wq_all[jj]
            wo_j = wo_all[jj]
            for b in range(B_LOC):
                q_all = jnp.dot(x_ref[b], wq_j,
                                preferred_element_type=jnp.float32)
                for h in range(H_LOC):
                    q = q_all[:, h * DH:(h + 1) * DH]
                    k = kv_k[slot, b, h]
                    v = kv_v[slot, b, h]
                    s = lax.dot_general(q, k, (((1,), (1,)), ((), ())),
                                        preferred_element_type=jnp.float32)
                    s = s + bias_ref[...]
                    w = jnp.exp(s)
                    w = w / jnp.sum(w, axis=1, keepdims=True)
                    ctx_buf[:, h * DH:(h + 1) * DH] = jnp.dot(
                        w, v, preferred_element_type=jnp.float32)
                acc = jnp.dot(ctx_buf[...], wo_j,
                              preferred_element_type=jnp.float32)
                if hop == 0:
                    out_ref[b] = acc
                else:
                    out_ref[b] = out_ref[b] + acc

        @functools.partial(pl.run_scoped,
                           exit_bar=pltpu.SemaphoreType.REGULAR)
        def _(exit_bar):
            for nbr in (left, right):
                pl.semaphore_signal(exit_bar, inc=1, device_id=(nbr,),
                                    device_id_type=pl.DeviceIdType.MESH)
            pl.semaphore_wait(exit_bar, 2)

    return pl.pallas_call(
        body,
        out_shape=jax.ShapeDtypeStruct((B_LOC, SQ, D), jnp.float32),
        in_specs=[
            pl.BlockSpec(memory_space=pltpu.VMEM),
            pl.BlockSpec(memory_space=pltpu.VMEM),
            pl.BlockSpec(memory_space=pltpu.MemorySpace.HBM),
            pl.BlockSpec(memory_space=pltpu.MemorySpace.HBM),
            pl.BlockSpec(memory_space=pltpu.VMEM),
        ],
        out_specs=pl.BlockSpec(memory_space=pltpu.VMEM),
        scratch_shapes=[
            pltpu.VMEM((W, D, H_LOC * DH), jnp.float32),
            pltpu.VMEM((W, H_LOC * DH, D), jnp.float32),
            pltpu.VMEM((2, B_LOC, H_LOC, SKV, DH), jnp.float32),
            pltpu.VMEM((2, B_LOC, H_LOC, SKV, DH), jnp.float32),
            pltpu.VMEM((SQ, SKV), jnp.float32),
            pltpu.VMEM((SQ, H_LOC * DH), jnp.float32),
            pltpu.SemaphoreType.DMA((2,)),
            pltpu.SemaphoreType.DMA((W - 1,)),
            pltpu.SemaphoreType.DMA((W - 1,)),
            pltpu.SemaphoreType.DMA((W - 1,)),
            pltpu.SemaphoreType.DMA((W - 1,)),
        ],
        compiler_params=pltpu.CompilerParams(
            collective_id=0, vmem_limit_bytes=100 * 1024 * 1024),
    )(x, Wq, K_ext, V_ext, Wo)


# device time: 228563 ns/iter; 3.8115x vs baseline; 3.2351x over previous
import functools

import jax
import jax.numpy as jnp
from jax import lax
from jax.experimental import pallas as pl
from jax.experimental.pallas import tpu as pltpu

W = 8
B_LOC = 2
SQ = 512
SKV = 512
H_LOC = 8
DH = 64
D = 768


def kernel(x, Wq, K_ext, V_ext, Wo):
    my = lax.axis_index("i")

    def arrange(ext):
        loc = lax.dynamic_slice_in_dim(ext, my * B_LOC, B_LOC, axis=0)
        loc = loc.astype(jnp.bfloat16)
        blocks = jnp.mod(my - jnp.arange(W), W)
        r = loc.reshape(B_LOC, SKV, W, H_LOC, DH)
        r = jnp.take(r, blocks, axis=2)
        return r.transpose(2, 3, 0, 1, 4)

    k_arr = arrange(K_ext)
    v_arr = arrange(V_ext)
    x_bf = x.astype(jnp.bfloat16)
    wq_bf = (Wq * 0.125).astype(jnp.bfloat16)
    wo_bf = Wo.astype(jnp.bfloat16)

    def body(x_ref, wq_ref, k_ref, v_ref, wo_ref, out_ref,
             wq_hops, wo_hops, bias_ref, ctx_buf,
             wq_ssem, wq_rsem, wo_ssem, wo_rsem):
        me = lax.axis_index("i")
        right = lax.rem(me + 1, W)
        left = lax.rem(me + W - 1, W)

        wq_hops[0] = wq_ref[...]
        wo_hops[0] = wo_ref[...]

        qi = lax.broadcasted_iota(jnp.int32, (SQ, SKV), 0)
        ki = lax.broadcasted_iota(jnp.int32, (SQ, SKV), 1)
        mask = (jnp.abs(qi - ki) <= 128) | (ki < 32) | (qi < 32)
        bias_ref[...] = jnp.where(mask, 0.0, -1e9).astype(jnp.float32)

        bar = pltpu.get_barrier_semaphore()
        for nbr in (left, right):
            pl.semaphore_signal(bar, inc=1, device_id=(nbr,),
                                device_id_type=pl.DeviceIdType.MESH)
        pl.semaphore_wait(bar, 2)

        prev_rdma = []
        for hop in range(W):
            for r in prev_rdma:
                r.wait()

            if hop < W - 1:
                r1 = pltpu.make_async_remote_copy(
                    src_ref=wq_hops.at[hop], dst_ref=wq_hops.at[hop + 1],
                    send_sem=wq_ssem.at[hop], recv_sem=wq_rsem.at[hop],
                    device_id=(right,), device_id_type=pl.DeviceIdType.MESH)
                r2 = pltpu.make_async_remote_copy(
                    src_ref=wo_hops.at[hop], dst_ref=wo_hops.at[hop + 1],
                    send_sem=wo_ssem.at[hop], recv_sem=wo_rsem.at[hop],
                    device_id=(right,), device_id_type=pl.DeviceIdType.MESH)
                r1.start()
                r2.start()
                prev_rdma = [r1, r2]

            wq_j = wq_hops[hop]
            wo_j = wo_hops[hop]
            for b in range(B_LOC):
                q_all = jnp.dot(x_ref[b], wq_j,
                                preferred_element_type=jnp.float32)
                q_bf = q_all.astype(jnp.bfloat16)
                for h in range(H_LOC):
                    q = q_bf[:, h * DH:(h + 1) * DH]
                    k = k_ref[hop, h, b]
                    v = v_ref[hop, h, b]
                    s = lax.dot_general(q, k, (((1,), (1,)), ((), ())),
                                        preferred_element_type=jnp.float32)
                    s = s + bias_ref[...]
                    w = jnp.exp(s)
                    w = (w / jnp.sum(w, axis=1, keepdims=True)).astype(
                        jnp.bfloat16)
                    ctx_buf[:, h * DH:(h + 1) * DH] = jnp.dot(
                        w, v, preferred_element_type=jnp.float32).astype(
                            jnp.bfloat16)
                acc = jnp.dot(ctx_buf[...], wo_j,
                              preferred_element_type=jnp.float32)
                if hop == 0:
                    out_ref[b] = acc
                else:
                    out_ref[b] = out_ref[b] + acc

        @functools.partial(pl.run_scoped,
                           exit_bar=pltpu.SemaphoreType.REGULAR)
        def _(exit_bar):
            for nbr in (left, right):
                pl.semaphore_signal(exit_bar, inc=1, device_id=(nbr,),
                                    device_id_type=pl.DeviceIdType.MESH)
            pl.semaphore_wait(exit_bar, 2)

    return pl.pallas_call(
        body,
        out_shape=jax.ShapeDtypeStruct((B_LOC, SQ, D), jnp.float32),
        in_specs=[
            pl.BlockSpec(memory_space=pltpu.VMEM),
            pl.BlockSpec(memory_space=pltpu.VMEM),
            pl.BlockSpec(memory_space=pltpu.VMEM),
            pl.BlockSpec(memory_space=pltpu.VMEM),
            pl.BlockSpec(memory_space=pltpu.VMEM),
        ],
        out_specs=pl.BlockSpec(memory_space=pltpu.VMEM),
        scratch_shapes=[
            pltpu.VMEM((W, D, H_LOC * DH), jnp.bfloat16),
            pltpu.VMEM((W, H_LOC * DH, D), jnp.bfloat16),
            pltpu.VMEM((SQ, SKV), jnp.float32),
            pltpu.VMEM((SQ, H_LOC * DH), jnp.bfloat16),
            pltpu.SemaphoreType.DMA((W - 1,)),
            pltpu.SemaphoreType.DMA((W - 1,)),
            pltpu.SemaphoreType.DMA((W - 1,)),
            pltpu.SemaphoreType.DMA((W - 1,)),
        ],
        compiler_params=pltpu.CompilerParams(
            collective_id=0, vmem_limit_bytes=60 * 1024 * 1024),
    )(x_bf, wq_bf, k_arr, v_arr, wo_bf)


# device time: 191073 ns/iter; 4.5593x vs baseline; 1.1962x over previous
import functools

import jax
import jax.numpy as jnp
from jax import lax
from jax.experimental import pallas as pl
from jax.experimental.pallas import tpu as pltpu

W = 8
B_LOC = 2
SQ = 512
SKV = 512
H_LOC = 8
DH = 64
D = 768


def kernel(x, Wq, K_ext, V_ext, Wo):
    my = lax.axis_index("i")

    def prep(ext):
        loc = lax.dynamic_slice_in_dim(ext, my * B_LOC, B_LOC, axis=0)
        return loc.astype(jnp.bfloat16).transpose(0, 2, 1, 3)

    k_loc = prep(K_ext)
    v_loc = prep(V_ext)
    x_bf = x.astype(jnp.bfloat16)
    wq_bf = (Wq * 0.125).astype(jnp.bfloat16)
    wo_bf = Wo.astype(jnp.bfloat16)

    def body(x_ref, wq_ref, k_ref, v_ref, wo_ref, out_ref,
             wq_hops, wo_hops, kv_k, kv_v, bias_ref, ctx_buf,
             kv_sems, wq_ssem, wq_rsem, wo_ssem, wo_rsem):
        me = lax.axis_index("i")
        right = lax.rem(me + 1, W)
        left = lax.rem(me + W - 1, W)

        wq_hops[0] = wq_ref[...]
        wo_hops[0] = wo_ref[...]

        def issue_kv(hop, slot):
            jj = lax.rem(me - hop + W, W)
            waits = []
            for b in range(B_LOC):
                for h in range(H_LOC):
                    hg = jj * H_LOC + h
                    for src, buf in ((k_ref, kv_k), (v_ref, kv_v)):
                        c = pltpu.make_async_copy(
                            src.at[b, hg], buf.at[slot, b, h],
                            kv_sems.at[slot])
                        c.start()
                        waits.append(c)
            return waits

        kv_waits = issue_kv(0, 0)

        qi = lax.broadcasted_iota(jnp.int32, (SQ, SKV), 0)
        ki = lax.broadcasted_iota(jnp.int32, (SQ, SKV), 1)
        mask = (jnp.abs(qi - ki) <= 128) | (ki < 32) | (qi < 32)
        bias_ref[...] = jnp.where(mask, 0.0, -1e9).astype(jnp.float32)

        bar = pltpu.get_barrier_semaphore()
        for nbr in (left, right):
            pl.semaphore_signal(bar, inc=1, device_id=(nbr,),
                                device_id_type=pl.DeviceIdType.MESH)
        pl.semaphore_wait(bar, 2)

        prev_rdma = []
        for hop in range(W):
            slot = hop % 2
            for r in prev_rdma:
                r.wait()

            if hop < W - 1:
                r1 = pltpu.make_async_remote_copy(
                    src_ref=wq_hops.at[hop], dst_ref=wq_hops.at[hop + 1],
                    send_sem=wq_ssem.at[hop], recv_sem=wq_rsem.at[hop],
                    device_id=(right,), device_id_type=pl.DeviceIdType.MESH)
                r2 = pltpu.make_async_remote_copy(
                    src_ref=wo_hops.at[hop], dst_ref=wo_hops.at[hop + 1],
                    send_sem=wo_ssem.at[hop], recv_sem=wo_rsem.at[hop],
                    device_id=(right,), device_id_type=pl.DeviceIdType.MESH)
                r1.start()
                r2.start()
                prev_rdma = [r1, r2]

            for c in kv_waits:
                c.wait()
            if hop < W - 1:
                kv_waits = issue_kv(hop + 1, 1 - slot)

            wq_j = wq_hops[hop]
            wo_j = wo_hops[hop]
            for b in range(B_LOC):
                q_all = jnp.dot(x_ref[b], wq_j,
                                preferred_element_type=jnp.float32
                                ).astype(jnp.bfloat16)
                for h in range(H_LOC):
                    q = q_all[:, h * DH:(h + 1) * DH]
                    k = kv_k[slot, b, h]
                    v = kv_v[slot, b, h]
                    s = lax.dot_general(q, k, (((1,), (1,)), ((), ())),
                                        preferred_element_type=jnp.float32)
                    w = jnp.exp(s + bias_ref[...])
                    denom = jnp.sum(w, axis=1, keepdims=True)
                    wb = w.astype(jnp.bfloat16)
                    ctx = jnp.dot(wb, v, preferred_element_type=jnp.float32)
                    ctx_buf[:, h * DH:(h + 1) * DH] = (
                        ctx / denom).astype(jnp.bfloat16)
                acc = jnp.dot(ctx_buf[...], wo_j,
                              preferred_element_type=jnp.float32)
                if hop == 0:
                    out_ref[b] = acc
                else:
                    out_ref[b] = out_ref[b] + acc

        @functools.partial(pl.run_scoped,
                           exit_bar=pltpu.SemaphoreType.REGULAR)
        def _(exit_bar):
            for nbr in (left, right):
                pl.semaphore_signal(exit_bar, inc=1, device_id=(nbr,),
                                    device_id_type=pl.DeviceIdType.MESH)
            pl.semaphore_wait(exit_bar, 2)

    return pl.pallas_call(
        body,
        out_shape=jax.ShapeDtypeStruct((B_LOC, SQ, D), jnp.float32),
        in_specs=[
            pl.BlockSpec(memory_space=pltpu.VMEM),
            pl.BlockSpec(memory_space=pltpu.VMEM),
            pl.BlockSpec(memory_space=pltpu.MemorySpace.HBM),
            pl.BlockSpec(memory_space=pltpu.MemorySpace.HBM),
            pl.BlockSpec(memory_space=pltpu.VMEM),
        ],
        out_specs=pl.BlockSpec(memory_space=pltpu.VMEM),
        scratch_shapes=[
            pltpu.VMEM((W, D, H_LOC * DH), jnp.bfloat16),
            pltpu.VMEM((W, H_LOC * DH, D), jnp.bfloat16),
            pltpu.VMEM((2, B_LOC, H_LOC, SKV, DH), jnp.bfloat16),
            pltpu.VMEM((2, B_LOC, H_LOC, SKV, DH), jnp.bfloat16),
            pltpu.VMEM((SQ, SKV), jnp.float32),
            pltpu.VMEM((SQ, H_LOC * DH), jnp.bfloat16),
            pltpu.SemaphoreType.DMA((2,)),
            pltpu.SemaphoreType.DMA((W - 1,)),
            pltpu.SemaphoreType.DMA((W - 1,)),
            pltpu.SemaphoreType.DMA((W - 1,)),
            pltpu.SemaphoreType.DMA((W - 1,)),
        ],
        compiler_params=pltpu.CompilerParams(
            collective_id=0, vmem_limit_bytes=60 * 1024 * 1024),
    )(x_bf, wq_bf, k_loc, v_loc, wo_bf)


# device time: 190871 ns/iter; 4.5642x vs baseline; 1.0011x over previous
import functools

import jax
import jax.numpy as jnp
from jax import lax
from jax.experimental import pallas as pl
from jax.experimental.pallas import tpu as pltpu

W = 8
B_LOC = 2
SQ = 512
SKV = 512
H_LOC = 8
DH = 64
D = 768


def kernel(x, Wq, K_ext, V_ext, Wo):
    my = lax.axis_index("i")

    def prep(ext):
        loc = lax.dynamic_slice_in_dim(ext, my * B_LOC, B_LOC, axis=0)
        return loc.astype(jnp.bfloat16).transpose(0, 2, 1, 3)

    k_loc = prep(K_ext)
    v_loc = prep(V_ext)
    x_bf = x.astype(jnp.bfloat16).reshape(B_LOC * SQ, D)
    wq_bf = (Wq * (0.125 * 1.4426950408889634)).astype(jnp.bfloat16)
    wo_bf = Wo.astype(jnp.bfloat16)

    def body(x_ref, wq_ref, k_ref, v_ref, wo_ref, out_ref,
             wq_hops, wo_hops, kv_k, kv_v, bias_ref, ctx_buf,
             kv_sems, wq_ssem, wq_rsem, wo_ssem, wo_rsem):
        me = lax.axis_index("i")
        right = lax.rem(me + 1, W)
        left = lax.rem(me + W - 1, W)

        wq_hops[0] = wq_ref[...]
        wo_hops[0] = wo_ref[...]

        def issue_kv(hop, slot):
            jj = lax.rem(me - hop + W, W)
            waits = []
            for b in range(B_LOC):
                for h in range(H_LOC):
                    hg = jj * H_LOC + h
                    for src, buf in ((k_ref, kv_k), (v_ref, kv_v)):
                        c = pltpu.make_async_copy(
                            src.at[b, hg], buf.at[slot, b, h],
                            kv_sems.at[slot])
                        c.start()
                        waits.append(c)
            return waits

        kv_waits = issue_kv(0, 0)

        qi = lax.broadcasted_iota(jnp.int32, (SQ, SKV), 0)
        ki = lax.broadcasted_iota(jnp.int32, (SQ, SKV), 1)
        mask = (jnp.abs(qi - ki) <= 128) | (ki < 32) | (qi < 32)
        bias_ref[...] = jnp.where(mask, 0.0, -1e9).astype(jnp.float32)

        bar = pltpu.get_barrier_semaphore()
        for nbr in (left, right):
            pl.semaphore_signal(bar, inc=1, device_id=(nbr,),
                                device_id_type=pl.DeviceIdType.MESH)
        pl.semaphore_wait(bar, 2)

        prev_rdma = []
        for hop in range(W):
            slot = hop % 2
            for r in prev_rdma:
                r.wait()

            if hop < W - 1:
                r1 = pltpu.make_async_remote_copy(
                    src_ref=wq_hops.at[hop], dst_ref=wq_hops.at[hop + 1],
                    send_sem=wq_ssem.at[hop], recv_sem=wq_rsem.at[hop],
                    device_id=(right,), device_id_type=pl.DeviceIdType.MESH)
                r2 = pltpu.make_async_remote_copy(
                    src_ref=wo_hops.at[hop], dst_ref=wo_hops.at[hop + 1],
                    send_sem=wo_ssem.at[hop], recv_sem=wo_rsem.at[hop],
                    device_id=(right,), device_id_type=pl.DeviceIdType.MESH)
                r1.start()
                r2.start()
                prev_rdma = [r1, r2]

            for c in kv_waits:
                c.wait()
            if hop < W - 1:
                kv_waits = issue_kv(hop + 1, 1 - slot)

            wq_j = wq_hops[hop]
            wo_j = wo_hops[hop]
            q_all = jnp.dot(x_ref[...], wq_j,
                            preferred_element_type=jnp.float32
                            ).astype(jnp.bfloat16)
            for b in range(B_LOC):
                for h in range(H_LOC):
                    q = q_all[b * SQ:(b + 1) * SQ, h * DH:(h + 1) * DH]
                    k = kv_k[slot, b, h]
                    v = kv_v[slot, b, h]
                    s = lax.dot_general(q, k, (((1,), (1,)), ((), ())),
                                        preferred_element_type=jnp.float32)
                    w = jnp.exp2(s + bias_ref[...])
                    denom = jnp.sum(w, axis=1, keepdims=True)
                    wb = w.astype(jnp.bfloat16)
                    ctx = jnp.dot(wb, v, preferred_element_type=jnp.float32)
                    ctx_buf[b * SQ:(b + 1) * SQ, h * DH:(h + 1) * DH] = (
                        ctx / denom).astype(jnp.bfloat16)
            acc = jnp.dot(ctx_buf[...], wo_j,
                          preferred_element_type=jnp.float32)
            if hop == 0:
                out_ref[...] = acc
            else:
                out_ref[...] = out_ref[...] + acc

        @functools.partial(pl.run_scoped,
                           exit_bar=pltpu.SemaphoreType.REGULAR)
        def _(exit_bar):
            for nbr in (left, right):
                pl.semaphore_signal(exit_bar, inc=1, device_id=(nbr,),
                                    device_id_type=pl.DeviceIdType.MESH)
            pl.semaphore_wait(exit_bar, 2)

    out = pl.pallas_call(
        body,
        out_shape=jax.ShapeDtypeStruct((B_LOC * SQ, D), jnp.float32),
        in_specs=[
            pl.BlockSpec(memory_space=pltpu.VMEM),
            pl.BlockSpec(memory_space=pltpu.VMEM),
            pl.BlockSpec(memory_space=pltpu.MemorySpace.HBM),
            pl.BlockSpec(memory_space=pltpu.MemorySpace.HBM),
            pl.BlockSpec(memory_space=pltpu.VMEM),
        ],
        out_specs=pl.BlockSpec(memory_space=pltpu.VMEM),
        scratch_shapes=[
            pltpu.VMEM((W, D, H_LOC * DH), jnp.bfloat16),
            pltpu.VMEM((W, H_LOC * DH, D), jnp.bfloat16),
            pltpu.VMEM((2, B_LOC, H_LOC, SKV, DH), jnp.bfloat16),
            pltpu.VMEM((2, B_LOC, H_LOC, SKV, DH), jnp.bfloat16),
            pltpu.VMEM((SQ, SKV), jnp.float32),
            pltpu.VMEM((B_LOC * SQ, H_LOC * DH), jnp.bfloat16),
            pltpu.SemaphoreType.DMA((2,)),
            pltpu.SemaphoreType.DMA((W - 1,)),
            pltpu.SemaphoreType.DMA((W - 1,)),
            pltpu.SemaphoreType.DMA((W - 1,)),
            pltpu.SemaphoreType.DMA((W - 1,)),
        ],
        compiler_params=pltpu.CompilerParams(
            collective_id=0, vmem_limit_bytes=60 * 1024 * 1024),
    )(x_bf, wq_bf, k_loc, v_loc, wo_bf)
    return out.reshape(B_LOC, SQ, D)


# device time: 172658 ns/iter; 5.0456x vs baseline; 1.1055x over previous
import functools

import jax
import jax.numpy as jnp
from jax import lax
from jax.experimental import pallas as pl
from jax.experimental.pallas import tpu as pltpu

W = 8
B_LOC = 2
SQ = 512
SKV = 512
H_LOC = 8
DH = 64
D = 768


def kernel(x, Wq, K_ext, V_ext, Wo):
    my = lax.axis_index("i")

    def prep(ext):
        loc = lax.dynamic_slice_in_dim(ext, my * B_LOC, B_LOC, axis=0)
        return loc.astype(jnp.bfloat16).transpose(0, 2, 3, 1)

    k_loc = prep(K_ext)
    v_loc = prep(V_ext)
    x_bf = x.astype(jnp.bfloat16).reshape(B_LOC * SQ, D)
    wq_bf = (Wq * (0.125 * 1.4426950408889634)).astype(jnp.bfloat16)
    wo_bf = Wo.astype(jnp.bfloat16)

    def body(x_ref, wq_ref, k_ref, v_ref, wo_ref, out_ref,
             wq_hops, wo_hops, kv_k, kv_v, bias_ref, ctx_buf,
             kv_sems, wq_ssem, wq_rsem, wo_ssem, wo_rsem):
        me = lax.axis_index("i")
        right = lax.rem(me + 1, W)
        left = lax.rem(me + W - 1, W)

        wq_hops[0] = wq_ref[...]
        wo_hops[0] = wo_ref[...]

        kv_k[...] = jnp.zeros(kv_k.shape, jnp.bfloat16)
        kv_v[...] = jnp.zeros(kv_v.shape, jnp.bfloat16)

        def issue_kv(hop, slot):
            jj = lax.rem(me - hop + W, W)
            waits = []
            for b in range(B_LOC):
                for p in range(H_LOC // 2):
                    for i in range(2):
                        hg = jj * H_LOC + 2 * p + i
                        blk = (slice(i * DH, (i + 1) * DH),
                               slice(i * SKV, (i + 1) * SKV))
                        for src, buf in ((k_ref, kv_k), (v_ref, kv_v)):
                            c = pltpu.make_async_copy(
                                src.at[b, hg],
                                buf.at[(slot, b, p) + blk],
                                kv_sems.at[slot])
                            c.start()
                            waits.append(c)
            return waits

        kv_waits = issue_kv(0, 0)

        qi = lax.broadcasted_iota(jnp.int32, (SQ, SKV), 0)
        ki = lax.broadcasted_iota(jnp.int32, (SQ, SKV), 1)
        mask = (jnp.abs(qi - ki) <= 128) | (ki < 32) | (qi < 32)
        b1 = jnp.where(mask, 0.0, -1e9).astype(jnp.float32)
        bias_ref[:, :SKV] = b1
        bias_ref[:, SKV:] = b1

        bar = pltpu.get_barrier_semaphore()
        for nbr in (left, right):
            pl.semaphore_signal(bar, inc=1, device_id=(nbr,),
                                device_id_type=pl.DeviceIdType.MESH)
        pl.semaphore_wait(bar, 2)

        prev_rdma = []
        for hop in range(W):
            slot = hop % 2
            for r in prev_rdma:
                r.wait()

            if hop < W - 1:
                r1 = pltpu.make_async_remote_copy(
                    src_ref=wq_hops.at[hop], dst_ref=wq_hops.at[hop + 1],
                    send_sem=wq_ssem.at[hop], recv_sem=wq_rsem.at[hop],
                    device_id=(right,), device_id_type=pl.DeviceIdType.MESH)
                r2 = pltpu.make_async_remote_copy(
                    src_ref=wo_hops.at[hop], dst_ref=wo_hops.at[hop + 1],
                    send_sem=wo_ssem.at[hop], recv_sem=wo_rsem.at[hop],
                    device_id=(right,), device_id_type=pl.DeviceIdType.MESH)
                r1.start()
                r2.start()
                prev_rdma = [r1, r2]

            for c in kv_waits:
                c.wait()
            if hop < W - 1:
                kv_waits = issue_kv(hop + 1, 1 - slot)

            wq_j = wq_hops[hop]
            wo_j = wo_hops[hop]
            q_all = jnp.dot(x_ref[...], wq_j,
                            preferred_element_type=jnp.float32
                            ).astype(jnp.bfloat16)
            for b in range(B_LOC):
                for p in range(H_LOC // 2):
                    q2 = q_all[b * SQ:(b + 1) * SQ,
                               p * 2 * DH:(p + 1) * 2 * DH]
                    kbd = kv_k[slot, b, p]
                    vbd = kv_v[slot, b, p]
                    s2 = jnp.dot(q2, kbd,
                                 preferred_element_type=jnp.float32)
                    w2 = jnp.exp2(s2 + bias_ref[...])
                    d0 = jnp.sum(w2[:, :SKV], axis=1, keepdims=True)
                    d1 = jnp.sum(w2[:, SKV:], axis=1, keepdims=True)
                    wb = w2.astype(jnp.bfloat16)
                    ctx2 = lax.dot_general(
                        wb, vbd, (((1,), (1,)), ((), ())),
                        preferred_element_type=jnp.float32)
                    col = b * SQ, p * 2 * DH
                    ctx_buf[col[0]:col[0] + SQ,
                            col[1]:col[1] + DH] = (
                        ctx2[:, :DH] / d0).astype(jnp.bfloat16)
                    ctx_buf[col[0]:col[0] + SQ,
                            col[1] + DH:col[1] + 2 * DH] = (
                        ctx2[:, DH:] / d1).astype(jnp.bfloat16)
            acc = jnp.dot(ctx_buf[...], wo_j,
                          preferred_element_type=jnp.float32)
            if hop == 0:
                out_ref[...] = acc
            else:
                out_ref[...] = out_ref[...] + acc

        @functools.partial(pl.run_scoped,
                           exit_bar=pltpu.SemaphoreType.REGULAR)
        def _(exit_bar):
            for nbr in (left, right):
                pl.semaphore_signal(exit_bar, inc=1, device_id=(nbr,),
                                    device_id_type=pl.DeviceIdType.MESH)
            pl.semaphore_wait(exit_bar, 2)

    out = pl.pallas_call(
        body,
        out_shape=jax.ShapeDtypeStruct((B_LOC * SQ, D), jnp.float32),
        in_specs=[
            pl.BlockSpec(memory_space=pltpu.VMEM),
            pl.BlockSpec(memory_space=pltpu.VMEM),
            pl.BlockSpec(memory_space=pltpu.MemorySpace.HBM),
            pl.BlockSpec(memory_space=pltpu.MemorySpace.HBM),
            pl.BlockSpec(memory_space=pltpu.VMEM),
        ],
        out_specs=pl.BlockSpec(memory_space=pltpu.VMEM),
        scratch_shapes=[
            pltpu.VMEM((W, D, H_LOC * DH), jnp.bfloat16),
            pltpu.VMEM((W, H_LOC * DH, D), jnp.bfloat16),
            pltpu.VMEM((2, B_LOC, H_LOC // 2, 2 * DH, 2 * SKV),
                       jnp.bfloat16),
            pltpu.VMEM((2, B_LOC, H_LOC // 2, 2 * DH, 2 * SKV),
                       jnp.bfloat16),
            pltpu.VMEM((SQ, 2 * SKV), jnp.float32),
            pltpu.VMEM((B_LOC * SQ, H_LOC * DH), jnp.bfloat16),
            pltpu.SemaphoreType.DMA((2,)),
            pltpu.SemaphoreType.DMA((W - 1,)),
            pltpu.SemaphoreType.DMA((W - 1,)),
            pltpu.SemaphoreType.DMA((W - 1,)),
            pltpu.SemaphoreType.DMA((W - 1,)),
        ],
        compiler_params=pltpu.CompilerParams(
            collective_id=0, vmem_limit_bytes=60 * 1024 * 1024),
    )(x_bf, wq_bf, k_loc, v_loc, wo_bf)
    return out.reshape(B_LOC, SQ, D)
